# baseline (speedup 1.0000x reference)
import functools
import math

import jax
import jax.numpy as jnp
import numpy as np
from jax import lax
from jax.experimental import pallas as pl
from jax.experimental.pallas import tpu as pltpu
from jax.experimental.pallas import tpu_sc as plsc

N_CLASSES = 100
D = 128
H = 256
B = 16384
HIST = 50
PADL = 56
NV = D // 16
NJ = D // 32
NBUF = 4

_PERM = np.concatenate(
    [np.concatenate([np.arange(32 * j, 32 * j + 32, 2),
                     np.arange(32 * j + 1, 32 * j + 32, 2)])
     for j in range(NJ)])


def _widen(w_i32):
    shift = jnp.full((16,), 16, jnp.int32)
    mask = jnp.full((16,), -65536, jnp.int32)
    even = plsc.bitcast(lax.shift_left(w_i32, shift), jnp.float32)
    odd = plsc.bitcast(lax.bitwise_and(w_i32, mask), jnp.float32)
    return even, odd


def _sc_sumsq(emb, x_pad_flat):
    info = plsc.get_sparse_core_info()
    NC, NS = info.num_cores, info.num_subcores
    NW = NC * NS
    b_per_w = B // NW
    idx_per_w = b_per_w * PADL
    mesh = plsc.VectorSubcoreMesh(core_axis_name="c", subcore_axis_name="s")

    @functools.partial(
        pl.kernel,
        out_type=jax.ShapeDtypeStruct((B, D), jnp.float32),
        mesh=mesh,
        scratch_types=[
            pltpu.VMEM((idx_per_w,), jnp.int32),
            pltpu.VMEM((HIST, D // 2), jnp.int32),
            pltpu.VMEM((HIST, D // 2), jnp.int32),
            pltpu.VMEM((HIST, D // 2), jnp.int32),
            pltpu.VMEM((HIST, D // 2), jnp.int32),
            pltpu.VMEM((b_per_w, D), jnp.float32),
            pltpu.SemaphoreType.DMA,
            pltpu.SemaphoreType.DMA,
            pltpu.SemaphoreType.DMA,
            pltpu.SemaphoreType.DMA,
        ],
        compiler_params=pltpu.CompilerParams(
            needs_layout_passes=False, use_tc_tiling_on_sc=False),
    )
    def k(emb_hbm, idx_hbm, out_hbm, idx_v, rows0, rows1, rows2, rows3,
          out_v, sem0, sem1, sem2, sem3):
        wid = lax.axis_index("s") * NC + lax.axis_index("c")
        rows = (rows0, rows1, rows2, rows3)
        sems = (sem0, sem1, sem2, sem3)
        pltpu.sync_copy(idx_hbm.at[pl.ds(wid * idx_per_w, idx_per_w)], idx_v)

        def gather_desc(c, b):
            return pltpu.make_async_copy(
                emb_hbm.at[idx_v.at[pl.ds(c * PADL, HIST)]],
                rows[b], sems[b])

        for p in range(NBUF):
            gather_desc(p, p).start()

        def chunk(c, b):
            gather_desc(c, b).wait()

            def body(r, accs, _b=b):
                out = []
                for j in range(NJ):
                    w = rows[_b][r, pl.ds(16 * j, 16)]
                    e, o = _widen(w)
                    out.append(accs[2 * j] + e * e)
                    out.append(accs[2 * j + 1] + o * o)
                return tuple(out)

            zeros = tuple(jnp.zeros((16,), jnp.float32) for _ in range(NV))
            res = lax.fori_loop(0, HIST, body, zeros, unroll=2)
            for v in range(NV):
                out_v[c, pl.ds(v * 16, 16)] = res[v]

            @pl.when(c + NBUF < b_per_w)
            def _():
                gather_desc(c + NBUF, b).start()

        def loop_body(i, carry):
            for b in range(NBUF):
                chunk(i * NBUF + b, b)
            return carry

        lax.fori_loop(0, b_per_w // NBUF, loop_body, 0)
        pltpu.sync_copy(out_v, out_hbm.at[pl.ds(wid * b_per_w, b_per_w)])

    return k(emb, x_pad_flat)


def _tc_mlp(s, W_in, b_in, W_h, b_h, W_out_p, b_out_p):
    BLK = 512

    def mlp(s_ref, wi, bi, wh, bh, wo, bo, o_ref):
        h = jnp.sqrt(s_ref[...] * float(D))
        h = jnp.maximum(
            jnp.dot(h, wi[...], preferred_element_type=jnp.float32) + bi[...],
            0.0)
        h = jnp.maximum(
            jnp.dot(h, wh[...], preferred_element_type=jnp.float32) + bh[...],
            0.0)
        o = jnp.dot(h, wo[...], preferred_element_type=jnp.float32) + bo[...]
        o_ref[...] = o[:, :N_CLASSES]

    return pl.pallas_call(
        mlp,
        grid=(B // BLK,),
        in_specs=[
            pl.BlockSpec((BLK, D), lambda i: (i, 0)),
            pl.BlockSpec((D, H), lambda i: (0, 0)),
            pl.BlockSpec((1, H), lambda i: (0, 0)),
            pl.BlockSpec((H, H), lambda i: (0, 0)),
            pl.BlockSpec((1, H), lambda i: (0, 0)),
            pl.BlockSpec((H, 128), lambda i: (0, 0)),
            pl.BlockSpec((1, 128), lambda i: (0, 0)),
        ],
        out_specs=pl.BlockSpec((BLK, N_CLASSES), lambda i: (i, 0)),
        out_shape=jax.ShapeDtypeStruct((B, N_CLASSES), jnp.float32),
    )(s, W_in, b_in.reshape(1, H), W_h, b_h.reshape(1, H),
      W_out_p, b_out_p)


def kernel(x, emb, W_in, b_in, W_h, b_h, W_out, b_out):
    x32 = x.astype(jnp.int32)
    x_pad = jnp.pad(x32, ((0, 0), (0, PADL - HIST)))
    emb_packed = lax.bitcast_convert_type(
        emb.astype(jnp.bfloat16).reshape(-1, D // 2, 2), jnp.int32)
    s = _sc_sumsq(emb_packed, x_pad.reshape(-1))
    W_in = W_in[jnp.asarray(_PERM), :]
    W_out_p = jnp.pad(W_out, ((0, 0), (0, 128 - N_CLASSES)))
    b_out_p = jnp.pad(b_out, (0, 128 - N_CLASSES)).reshape(1, 128)
    return _tc_mlp(s, W_in, b_in, W_h, b_h, W_out_p, b_out_p)


# speedup vs baseline: 2.6982x; 2.6982x over previous
import functools
import math

import jax
import jax.numpy as jnp
from jax import lax
from jax.experimental import pallas as pl
from jax.experimental.pallas import tpu as pltpu
from jax.experimental.pallas import tpu_sc as plsc

N_CLASSES = 100
D = 128
H = 256
B = 16384
HIST = 50
PADL = 56
NV = D // 16
NBUF = 4


def _sc_sumsq(emb, x_pad_flat):
    info = plsc.get_sparse_core_info()
    NC, NS = info.num_cores, info.num_subcores
    NW = NC * NS
    b_per_w = B // NW
    idx_per_w = b_per_w * PADL
    mesh = plsc.VectorSubcoreMesh(core_axis_name="c", subcore_axis_name="s")

    @functools.partial(
        pl.kernel,
        out_type=jax.ShapeDtypeStruct((B, D), jnp.float32),
        mesh=mesh,
        scratch_types=[
            pltpu.VMEM((idx_per_w,), jnp.int32),
            pltpu.VMEM((HIST, D), jnp.float32),
            pltpu.VMEM((HIST, D), jnp.float32),
            pltpu.VMEM((HIST, D), jnp.float32),
            pltpu.VMEM((HIST, D), jnp.float32),
            pltpu.VMEM((b_per_w, D), jnp.float32),
            pltpu.SemaphoreType.DMA,
            pltpu.SemaphoreType.DMA,
            pltpu.SemaphoreType.DMA,
            pltpu.SemaphoreType.DMA,
        ],
        compiler_params=pltpu.CompilerParams(needs_layout_passes=False),
    )
    def k(emb_hbm, idx_hbm, out_hbm, idx_v, rows0, rows1, rows2, rows3,
          out_v, sem0, sem1, sem2, sem3):
        wid = lax.axis_index("s") * NC + lax.axis_index("c")
        rows = (rows0, rows1, rows2, rows3)
        sems = (sem0, sem1, sem2, sem3)
        pltpu.sync_copy(idx_hbm.at[pl.ds(wid * idx_per_w, idx_per_w)], idx_v)

        def gather_desc(c, b):
            return pltpu.make_async_copy(
                emb_hbm.at[idx_v.at[pl.ds(c * PADL, HIST)]],
                rows[b], sems[b])

        for p in range(NBUF):
            gather_desc(p, p).start()

        def chunk(c, b):
            gather_desc(c, b).wait()

            def body(r, accs, _b=b):
                vals = [rows[_b][r, pl.ds(v * 16, 16)] for v in range(NV)]
                return tuple(accs[v] + vals[v] * vals[v] for v in range(NV))

            zeros = tuple(jnp.zeros((16,), jnp.float32) for _ in range(NV))
            res = lax.fori_loop(0, HIST, body, zeros, unroll=2)
            for v in range(NV):
                out_v[c, pl.ds(v * 16, 16)] = res[v]

            @pl.when(c + NBUF < b_per_w)
            def _():
                gather_desc(c + NBUF, b).start()

        def loop_body(i, carry):
            for b in range(NBUF):
                chunk(i * NBUF + b, b)
            return carry

        lax.fori_loop(0, b_per_w // NBUF, loop_body, 0)
        pltpu.sync_copy(out_v, out_hbm.at[pl.ds(wid * b_per_w, b_per_w)])

    return k(emb, x_pad_flat)


def _tc_mlp(s, W_in, b_in, W_h, b_h, W_out_p, b_out_p):
    BLK = 512

    def mlp(s_ref, wi, bi, wh, bh, wo, bo, o_ref):
        h = jnp.sqrt(s_ref[...] * float(D)).astype(jnp.bfloat16)
        h = jnp.maximum(
            jnp.dot(h, wi[...], preferred_element_type=jnp.float32) + bi[...],
            0.0).astype(jnp.bfloat16)
        h = jnp.maximum(
            jnp.dot(h, wh[...], preferred_element_type=jnp.float32) + bh[...],
            0.0).astype(jnp.bfloat16)
        o = jnp.dot(h, wo[...], preferred_element_type=jnp.float32) + bo[...]
        o_ref[...] = o[:, :N_CLASSES]

    return pl.pallas_call(
        mlp,
        grid=(B // BLK,),
        in_specs=[
            pl.BlockSpec((BLK, D), lambda i: (i, 0)),
            pl.BlockSpec((D, H), lambda i: (0, 0)),
            pl.BlockSpec((1, H), lambda i: (0, 0)),
            pl.BlockSpec((H, H), lambda i: (0, 0)),
            pl.BlockSpec((1, H), lambda i: (0, 0)),
            pl.BlockSpec((H, 128), lambda i: (0, 0)),
            pl.BlockSpec((1, 128), lambda i: (0, 0)),
        ],
        out_specs=pl.BlockSpec((BLK, N_CLASSES), lambda i: (i, 0)),
        out_shape=jax.ShapeDtypeStruct((B, N_CLASSES), jnp.float32),
    )(s, W_in.astype(jnp.bfloat16), b_in.reshape(1, H),
      W_h.astype(jnp.bfloat16), b_h.reshape(1, H),
      W_out_p.astype(jnp.bfloat16), b_out_p)


def kernel(x, emb, W_in, b_in, W_h, b_h, W_out, b_out):
    x32 = x.astype(jnp.int32)
    x_pad = jnp.pad(x32, ((0, 0), (0, PADL - HIST)))
    s = _sc_sumsq(emb, x_pad.reshape(-1))
    W_out_p = jnp.pad(W_out, ((0, 0), (0, 128 - N_CLASSES)))
    b_out_p = jnp.pad(b_out, (0, 128 - N_CLASSES)).reshape(1, 128)
    return _tc_mlp(s, W_in, b_in, W_h, b_h, W_out_p, b_out_p)


# speedup vs baseline: 2.8522x; 1.0571x over previous
import functools
import math

import jax
import jax.numpy as jnp
from jax import lax
from jax.experimental import pallas as pl
from jax.experimental.pallas import tpu as pltpu
from jax.experimental.pallas import tpu_sc as plsc

N_CLASSES = 100
D = 128
H = 256
B = 16384
HIST = 50
PADL = 56
NV = D // 16
NBUF = 4


def _sc_sumsq(emb, x_pad_flat):
    info = plsc.get_sparse_core_info()
    NC, NS = info.num_cores, info.num_subcores
    NW = NC * NS
    b_per_w = B // NW
    idx_per_w = b_per_w * PADL
    mesh = plsc.VectorSubcoreMesh(core_axis_name="c", subcore_axis_name="s")

    @functools.partial(
        pl.kernel,
        out_type=jax.ShapeDtypeStruct((B, D), jnp.float32),
        mesh=mesh,
        scratch_types=[
            pltpu.VMEM((idx_per_w,), jnp.int32),
            pltpu.VMEM((HIST, D), jnp.float32),
            pltpu.VMEM((HIST, D), jnp.float32),
            pltpu.VMEM((HIST, D), jnp.float32),
            pltpu.VMEM((HIST, D), jnp.float32),
            pltpu.VMEM((b_per_w, D), jnp.float32),
            pltpu.SemaphoreType.DMA,
            pltpu.SemaphoreType.DMA,
            pltpu.SemaphoreType.DMA,
            pltpu.SemaphoreType.DMA,
        ],
        compiler_params=pltpu.CompilerParams(needs_layout_passes=False),
    )
    def k(emb_hbm, idx_hbm, out_hbm, idx_v, rows0, rows1, rows2, rows3,
          out_v, sem0, sem1, sem2, sem3):
        wid = lax.axis_index("s") * NC + lax.axis_index("c")
        rows = (rows0, rows1, rows2, rows3)
        sems = (sem0, sem1, sem2, sem3)
        pltpu.sync_copy(idx_hbm.at[pl.ds(wid * idx_per_w, idx_per_w)], idx_v)

        def gather_desc(c, b):
            return pltpu.make_async_copy(
                emb_hbm.at[idx_v.at[pl.ds(c * PADL, HIST)]],
                rows[b], sems[b])

        for p in range(NBUF):
            gather_desc(p, p).start()

        def chunk(c, b):
            gather_desc(c, b).wait()

            def body(r, accs, _b=b):
                vals = [rows[_b][r, pl.ds(v * 16, 16)] for v in range(NV)]
                return tuple(accs[v] + vals[v] * vals[v] for v in range(NV))

            zeros = tuple(jnp.zeros((16,), jnp.float32) for _ in range(NV))
            res = lax.fori_loop(0, HIST, body, zeros, unroll=2)
            for v in range(NV):
                out_v[c, pl.ds(v * 16, 16)] = res[v]

            @pl.when(c + NBUF < b_per_w)
            def _():
                gather_desc(c + NBUF, b).start()

        def loop_body(i, carry):
            for b in range(NBUF):
                chunk(i * NBUF + b, b)
            return carry

        lax.fori_loop(0, b_per_w // NBUF, loop_body, 0)
        pltpu.sync_copy(out_v, out_hbm.at[pl.ds(wid * b_per_w, b_per_w)])

    return k(emb, x_pad_flat)


def _tc_mlp(s, W_in, b_in, W_h, b_h, W_out_p, b_out_p):
    BLK = 2048

    def mlp(s_ref, wi, bi, wh, bh, wo, bo, o_ref):
        h = jnp.sqrt(s_ref[...] * float(D))
        h = jnp.maximum(
            jnp.dot(h, wi[...], preferred_element_type=jnp.float32) + bi[...],
            0.0)
        h = jnp.maximum(
            jnp.dot(h, wh[...], preferred_element_type=jnp.float32) + bh[...],
            0.0)
        o = jnp.dot(h, wo[...], preferred_element_type=jnp.float32) + bo[...]
        o_ref[...] = o[:, :N_CLASSES]

    return pl.pallas_call(
        mlp,
        grid=(B // BLK,),
        in_specs=[
            pl.BlockSpec((BLK, D), lambda i: (i, 0)),
            pl.BlockSpec((D, H), lambda i: (0, 0)),
            pl.BlockSpec((1, H), lambda i: (0, 0)),
            pl.BlockSpec((H, H), lambda i: (0, 0)),
            pl.BlockSpec((1, H), lambda i: (0, 0)),
            pl.BlockSpec((H, 128), lambda i: (0, 0)),
            pl.BlockSpec((1, 128), lambda i: (0, 0)),
        ],
        out_specs=pl.BlockSpec((BLK, N_CLASSES), lambda i: (i, 0)),
        out_shape=jax.ShapeDtypeStruct((B, N_CLASSES), jnp.float32),
    )(s, W_in, b_in.reshape(1, H), W_h, b_h.reshape(1, H),
      W_out_p, b_out_p)


def kernel(x, emb, W_in, b_in, W_h, b_h, W_out, b_out):
    x32 = x.astype(jnp.int32)
    x_pad = jnp.pad(x32, ((0, 0), (0, PADL - HIST)))
    s = _sc_sumsq(emb, x_pad.reshape(-1))
    W_out_p = jnp.pad(W_out, ((0, 0), (0, 128 - N_CLASSES)))
    b_out_p = jnp.pad(b_out, (0, 128 - N_CLASSES)).reshape(1, 128)
    return _tc_mlp(s, W_in, b_in, W_h, b_h, W_out_p, b_out_p)


# speedup vs baseline: 2.8674x; 1.0053x over previous
import functools
import math

import jax
import jax.numpy as jnp
from jax import lax
from jax.experimental import pallas as pl
from jax.experimental.pallas import tpu as pltpu
from jax.experimental.pallas import tpu_sc as plsc

N_CLASSES = 100
D = 128
H = 256
B = 16384
HIST = 50
PADL = 56
NV = D // 16
NBUF = 4


def _sc_sumsq(emb, x_pad_flat):
    info = plsc.get_sparse_core_info()
    NC, NS = info.num_cores, info.num_subcores
    NW = NC * NS
    b_per_w = B // NW
    idx_per_w = b_per_w * PADL
    mesh = plsc.VectorSubcoreMesh(core_axis_name="c", subcore_axis_name="s")

    @functools.partial(
        pl.kernel,
        out_type=jax.ShapeDtypeStruct((B, D), jnp.float32),
        mesh=mesh,
        scratch_types=[
            pltpu.VMEM((idx_per_w,), jnp.int32),
            pltpu.VMEM((HIST, D), jnp.float32),
            pltpu.VMEM((HIST, D), jnp.float32),
            pltpu.VMEM((HIST, D), jnp.float32),
            pltpu.VMEM((HIST, D), jnp.float32),
            pltpu.VMEM((b_per_w, D), jnp.float32),
            pltpu.SemaphoreType.DMA,
            pltpu.SemaphoreType.DMA,
            pltpu.SemaphoreType.DMA,
            pltpu.SemaphoreType.DMA,
        ],
        compiler_params=pltpu.CompilerParams(needs_layout_passes=False),
    )
    def k(emb_hbm, idx_hbm, out_hbm, idx_v, rows0, rows1, rows2, rows3,
          out_v, sem0, sem1, sem2, sem3):
        wid = lax.axis_index("s") * NC + lax.axis_index("c")
        rows = (rows0, rows1, rows2, rows3)
        sems = (sem0, sem1, sem2, sem3)
        pltpu.sync_copy(idx_hbm.at[pl.ds(wid * idx_per_w, idx_per_w)], idx_v)

        def gather_desc(c, b):
            return pltpu.make_async_copy(
                emb_hbm.at[idx_v.at[pl.ds(c * PADL, HIST)]],
                rows[b], sems[b])

        for p in range(NBUF):
            gather_desc(p, p).start()

        def chunk(c, b):
            gather_desc(c, b).wait()

            def body(r, accs, _b=b):
                vals = [rows[_b][r, pl.ds(v * 16, 16)] for v in range(NV)]
                return tuple(accs[v] + vals[v] * vals[v] for v in range(NV))

            zeros = tuple(jnp.zeros((16,), jnp.float32) for _ in range(NV))
            res = lax.fori_loop(0, HIST, body, zeros, unroll=2)
            for v in range(NV):
                out_v[c, pl.ds(v * 16, 16)] = res[v]

            @pl.when(c + NBUF < b_per_w)
            def _():
                gather_desc(c + NBUF, b).start()

        def loop_body(i, carry):
            for b in range(NBUF):
                chunk(i * NBUF + b, b)
            return carry

        lax.fori_loop(0, b_per_w // NBUF, loop_body, 0)
        pltpu.sync_copy(out_v, out_hbm.at[pl.ds(wid * b_per_w, b_per_w)])

    return k(emb, x_pad_flat)


def _tc_mlp(s, W_in, b_in, W_h, b_h, W_out_p, b_out_p):
    BLK = 4096

    def mlp(s_ref, wi, bi, wh, bh, wo, bo, o_ref):
        h = jnp.sqrt(s_ref[...] * float(D))
        h = jnp.maximum(
            jnp.dot(h, wi[...], preferred_element_type=jnp.float32) + bi[...],
            0.0)
        h = jnp.maximum(
            jnp.dot(h, wh[...], preferred_element_type=jnp.float32) + bh[...],
            0.0)
        o = jnp.dot(h, wo[...], preferred_element_type=jnp.float32) + bo[...]
        o_ref[...] = o[:, :N_CLASSES]

    return pl.pallas_call(
        mlp,
        grid=(B // BLK,),
        in_specs=[
            pl.BlockSpec((BLK, D), lambda i: (i, 0)),
            pl.BlockSpec((D, H), lambda i: (0, 0)),
            pl.BlockSpec((1, H), lambda i: (0, 0)),
            pl.BlockSpec((H, H), lambda i: (0, 0)),
            pl.BlockSpec((1, H), lambda i: (0, 0)),
            pl.BlockSpec((H, 128), lambda i: (0, 0)),
            pl.BlockSpec((1, 128), lambda i: (0, 0)),
        ],
        out_specs=pl.BlockSpec((BLK, N_CLASSES), lambda i: (i, 0)),
        out_shape=jax.ShapeDtypeStruct((B, N_CLASSES), jnp.float32),
    )(s, W_in, b_in.reshape(1, H), W_h, b_h.reshape(1, H),
      W_out_p, b_out_p)


def kernel(x, emb, W_in, b_in, W_h, b_h, W_out, b_out):
    x32 = x.astype(jnp.int32)
    x_pad = jnp.pad(x32, ((0, 0), (0, PADL - HIST)))
    s = _sc_sumsq(emb, x_pad.reshape(-1))
    W_out_p = jnp.pad(W_out, ((0, 0), (0, 128 - N_CLASSES)))
    b_out_p = jnp.pad(b_out, (0, 128 - N_CLASSES)).reshape(1, 128)
    return _tc_mlp(s, W_in, b_in, W_h, b_h, W_out_p, b_out_p)


# speedup vs baseline: 2.8714x; 1.0014x over previous
import functools
import math

import jax
import jax.numpy as jnp
from jax import lax
from jax.experimental import pallas as pl
from jax.experimental.pallas import tpu as pltpu
from jax.experimental.pallas import tpu_sc as plsc

N_CLASSES = 100
D = 128
H = 256
B = 16384
HIST = 50
PADL = 56
NV = D // 16
NBUF = 4


def _sc_sumsq(emb, x_pad_flat):
    info = plsc.get_sparse_core_info()
    NC, NS = info.num_cores, info.num_subcores
    NW = NC * NS
    b_per_w = B // NW
    idx_per_w = b_per_w * PADL
    mesh = plsc.VectorSubcoreMesh(core_axis_name="c", subcore_axis_name="s")

    @functools.partial(
        pl.kernel,
        out_type=jax.ShapeDtypeStruct((B * D,), jnp.float32),
        mesh=mesh,
        scratch_types=[
            pltpu.VMEM((idx_per_w,), jnp.int32),
            pltpu.VMEM((HIST, D), jnp.float32),
            pltpu.VMEM((HIST, D), jnp.float32),
            pltpu.VMEM((HIST, D), jnp.float32),
            pltpu.VMEM((HIST, D), jnp.float32),
            pltpu.VMEM((b_per_w * D,), jnp.float32),
            pltpu.SemaphoreType.DMA,
            pltpu.SemaphoreType.DMA,
            pltpu.SemaphoreType.DMA,
            pltpu.SemaphoreType.DMA,
        ],
        compiler_params=pltpu.CompilerParams(needs_layout_passes=False),
    )
    def k(emb_hbm, idx_hbm, out_hbm, idx_v, rows0, rows1, rows2, rows3,
          out_v, sem0, sem1, sem2, sem3):
        wid = lax.axis_index("s") * NC + lax.axis_index("c")
        rows = (rows0, rows1, rows2, rows3)
        sems = (sem0, sem1, sem2, sem3)
        pltpu.sync_copy(idx_hbm.at[pl.ds(wid * idx_per_w, idx_per_w)], idx_v)

        def gather_desc(c, b):
            return pltpu.make_async_copy(
                emb_hbm.at[idx_v.at[pl.ds(c * PADL, HIST)]],
                rows[b], sems[b])

        for p in range(NBUF):
            gather_desc(p, p).start()

        def chunk(c, b):
            gather_desc(c, b).wait()

            def body(r, accs, _b=b):
                vals = [rows[_b][r, pl.ds(v * 16, 16)] for v in range(NV)]
                return tuple(accs[v] + vals[v] * vals[v] for v in range(NV))

            zeros = tuple(jnp.zeros((16,), jnp.float32) for _ in range(NV))
            res = lax.fori_loop(0, HIST, body, zeros, unroll=2)
            for v in range(NV):
                out_v[pl.ds(c * D + v * 16, 16)] = res[v]

            @pl.when(c + NBUF < b_per_w)
            def _():
                gather_desc(c + NBUF, b).start()

        def loop_body(i, carry):
            for b in range(NBUF):
                chunk(i * NBUF + b, b)
            return carry

        lax.fori_loop(0, b_per_w // NBUF, loop_body, 0)
        pltpu.sync_copy(
            out_v, out_hbm.at[pl.ds(wid * b_per_w * D, b_per_w * D)])

    return k(emb, x_pad_flat)


def _tc_mlp(s, W_in, b_in, W_h, b_h, W_out_p, b_out_p):
    BLK = 4096

    def mlp(s_ref, wi, bi, wh, bh, wo, bo, o_ref):
        h = jnp.sqrt(s_ref[...].reshape(BLK, D) * float(D))
        h = jnp.maximum(
            jnp.dot(h, wi[...], preferred_element_type=jnp.float32) + bi[...],
            0.0)
        h = jnp.maximum(
            jnp.dot(h, wh[...], preferred_element_type=jnp.float32) + bh[...],
            0.0)
        o = jnp.dot(h, wo[...], preferred_element_type=jnp.float32) + bo[...]
        o_ref[...] = o[:, :N_CLASSES]

    return pl.pallas_call(
        mlp,
        grid=(B // BLK,),
        in_specs=[
            pl.BlockSpec((BLK * D,), lambda i: (i,)),
            pl.BlockSpec((D, H), lambda i: (0, 0)),
            pl.BlockSpec((1, H), lambda i: (0, 0)),
            pl.BlockSpec((H, H), lambda i: (0, 0)),
            pl.BlockSpec((1, H), lambda i: (0, 0)),
            pl.BlockSpec((H, 128), lambda i: (0, 0)),
            pl.BlockSpec((1, 128), lambda i: (0, 0)),
        ],
        out_specs=pl.BlockSpec((BLK, N_CLASSES), lambda i: (i, 0)),
        out_shape=jax.ShapeDtypeStruct((B, N_CLASSES), jnp.float32),
    )(s, W_in, b_in.reshape(1, H), W_h, b_h.reshape(1, H),
      W_out_p, b_out_p)


def kernel(x, emb, W_in, b_in, W_h, b_h, W_out, b_out):
    x32 = x.astype(jnp.int32)
    x_pad = jnp.pad(x32, ((0, 0), (0, PADL - HIST)))
    s = _sc_sumsq(emb, x_pad.reshape(-1))
    W_out_p = jnp.pad(W_out, ((0, 0), (0, 128 - N_CLASSES)))
    b_out_p = jnp.pad(b_out, (0, 128 - N_CLASSES)).reshape(1, 128)
    return _tc_mlp(s, W_in, b_in, W_h, b_h, W_out_p, b_out_p)


# speedup vs baseline: 2.9850x; 1.0396x over previous
import functools
import math

import jax
import jax.numpy as jnp
from jax import lax
from jax.experimental import pallas as pl
from jax.experimental.pallas import tpu as pltpu
from jax.experimental.pallas import tpu_sc as plsc

N_CLASSES = 100
D = 128
H = 256
B = 16384
HIST = 50
PADL = 56
NV = D // 16
NBUF = 4


def _sc_sumsq(emb, x_pad):
    info = plsc.get_sparse_core_info()
    NC, NS = info.num_cores, info.num_subcores
    NW = NC * NS
    b_per_w = B // NW
    mesh = plsc.VectorSubcoreMesh(core_axis_name="c", subcore_axis_name="s")

    @functools.partial(
        pl.kernel,
        out_type=jax.ShapeDtypeStruct((B, D), jnp.float32),
        mesh=mesh,
        scratch_types=[
            pltpu.VMEM((b_per_w, 128), jnp.int32),
            pltpu.VMEM((HIST, D), jnp.float32),
            pltpu.VMEM((HIST, D), jnp.float32),
            pltpu.VMEM((HIST, D), jnp.float32),
            pltpu.VMEM((HIST, D), jnp.float32),
            pltpu.VMEM((64, D), jnp.float32),
            pltpu.SemaphoreType.DMA,
            pltpu.SemaphoreType.DMA,
            pltpu.SemaphoreType.DMA,
            pltpu.SemaphoreType.DMA,
        ],
        compiler_params=pltpu.CompilerParams(needs_layout_passes=False),
    )
    def k(emb_hbm, idx_hbm, out_hbm, idx_v, rows0, rows1, rows2, rows3,
          out_v, sem0, sem1, sem2, sem3):
        wid = lax.axis_index("s") * NC + lax.axis_index("c")
        rows = (rows0, rows1, rows2, rows3)
        sems = (sem0, sem1, sem2, sem3)
        pltpu.sync_copy(idx_hbm.at[pl.ds(wid * b_per_w, b_per_w)], idx_v)

        def gather_desc(c, b):
            return pltpu.make_async_copy(
                emb_hbm.at[idx_v.at[c, pl.ds(0, HIST)]],
                rows[b], sems[b])

        for p in range(NBUF):
            gather_desc(p, p).start()

        def chunk(c, b):
            gather_desc(c, b).wait()

            def body(r, accs, _b=b):
                vals = [rows[_b][r, pl.ds(v * 16, 16)] for v in range(NV)]
                return tuple(accs[v] + vals[v] * vals[v] for v in range(NV))

            zeros = tuple(jnp.zeros((16,), jnp.float32) for _ in range(NV))
            res = lax.fori_loop(0, HIST, body, zeros, unroll=2)
            orow = lax.rem(c, 64)
            for v in range(NV):
                out_v[orow, pl.ds(v * 16, 16)] = res[v]

            @pl.when(c + NBUF < b_per_w)
            def _():
                gather_desc(c + NBUF, b).start()

            @pl.when(lax.rem(c, 64) == 63)
            def _():
                base = pl.multiple_of(wid * b_per_w + c - 63, 64)
                pltpu.sync_copy(out_v, out_hbm.at[pl.ds(base, 64)])

        def loop_body(i, carry):
            for b in range(NBUF):
                chunk(i * NBUF + b, b)
            return carry

        lax.fori_loop(0, b_per_w // NBUF, loop_body, 0)

    return k(emb, x_pad)


def _tc_mlp(s, W_in, b_in, W_h, b_h, W_out_p, b_out_p):
    BLK = 4096

    def mlp(s_ref, wi, bi, wh, bh, wo, bo, o_ref):
        h = jnp.sqrt(s_ref[...] * float(D))
        h = jnp.maximum(
            jnp.dot(h, wi[...], preferred_element_type=jnp.float32) + bi[...],
            0.0)
        h = jnp.maximum(
            jnp.dot(h, wh[...], preferred_element_type=jnp.float32) + bh[...],
            0.0)
        o = jnp.dot(h, wo[...], preferred_element_type=jnp.float32) + bo[...]
        o_ref[...] = o[:, :N_CLASSES]

    return pl.pallas_call(
        mlp,
        grid=(B // BLK,),
        in_specs=[
            pl.BlockSpec((BLK, D), lambda i: (i, 0)),
            pl.BlockSpec((D, H), lambda i: (0, 0)),
            pl.BlockSpec((1, H), lambda i: (0, 0)),
            pl.BlockSpec((H, H), lambda i: (0, 0)),
            pl.BlockSpec((1, H), lambda i: (0, 0)),
            pl.BlockSpec((H, 128), lambda i: (0, 0)),
            pl.BlockSpec((1, 128), lambda i: (0, 0)),
        ],
        out_specs=pl.BlockSpec((BLK, N_CLASSES), lambda i: (i, 0)),
        out_shape=jax.ShapeDtypeStruct((B, N_CLASSES), jnp.float32),
    )(s, W_in, b_in.reshape(1, H), W_h, b_h.reshape(1, H),
      W_out_p, b_out_p)


def kernel(x, emb, W_in, b_in, W_h, b_h, W_out, b_out):
    x32 = x.astype(jnp.int32)
    x_pad = jnp.pad(x32, ((0, 0), (0, 128 - HIST)))
    s = _sc_sumsq(emb, x_pad)
    W_out_p = jnp.pad(W_out, ((0, 0), (0, 128 - N_CLASSES)))
    b_out_p = jnp.pad(b_out, (0, 128 - N_CLASSES)).reshape(1, 128)
    return _tc_mlp(s, W_in, b_in, W_h, b_h, W_out_p, b_out_p)


# speedup vs baseline: 2.9869x; 1.0006x over previous
import functools

import jax
import jax.numpy as jnp
from jax import lax
from jax.experimental import pallas as pl
from jax.experimental.pallas import tpu as pltpu
from jax.experimental.pallas import tpu_sc as plsc

N_CLASSES = 100
D = 128
H = 256
B = 16384
HIST = 50
NV = D // 16
NBUF = 4


def _sc_sumsq(emb, x_pad):
    info = plsc.get_sparse_core_info()
    NC, NS = info.num_cores, info.num_subcores
    NW = NC * NS
    b_per_w = B // NW
    mesh = plsc.VectorSubcoreMesh(core_axis_name="c", subcore_axis_name="s")

    @functools.partial(
        pl.kernel,
        out_type=jax.ShapeDtypeStruct((B, D), jnp.float32),
        mesh=mesh,
        scratch_types=[
            pltpu.VMEM((b_per_w, 128), jnp.int32),
            pltpu.VMEM((HIST, D), jnp.float32),
            pltpu.VMEM((HIST, D), jnp.float32),
            pltpu.VMEM((HIST, D), jnp.float32),
            pltpu.VMEM((HIST, D), jnp.float32),
            pltpu.VMEM((64, D), jnp.float32),
            pltpu.SemaphoreType.DMA,
            pltpu.SemaphoreType.DMA,
            pltpu.SemaphoreType.DMA,
            pltpu.SemaphoreType.DMA,
        ],
        compiler_params=pltpu.CompilerParams(needs_layout_passes=False),
    )
    def k(emb_hbm, idx_hbm, out_hbm, idx_v, rows0, rows1, rows2, rows3,
          out_v, sem0, sem1, sem2, sem3):
        wid = lax.axis_index("s") * NC + lax.axis_index("c")
        rows = (rows0, rows1, rows2, rows3)
        sems = (sem0, sem1, sem2, sem3)
        pltpu.sync_copy(idx_hbm.at[pl.ds(wid * b_per_w, b_per_w)], idx_v)

        def gather_desc(c, b):
            return pltpu.make_async_copy(
                emb_hbm.at[idx_v.at[c, pl.ds(0, HIST)]],
                rows[b], sems[b])

        for p in range(NBUF):
            gather_desc(p, p).start()

        def chunk(c, b):
            gather_desc(c, b).wait()

            def body(r, accs, _b=b):
                vals = [rows[_b][r, pl.ds(v * 16, 16)] for v in range(NV)]
                return tuple(accs[v] + vals[v] * vals[v] for v in range(NV))

            zeros = tuple(jnp.zeros((16,), jnp.float32) for _ in range(NV))
            res = lax.fori_loop(0, HIST, body, zeros, unroll=2)
            orow = lax.rem(c, 64)
            for v in range(NV):
                out_v[orow, pl.ds(v * 16, 16)] = res[v]

            @pl.when(c + NBUF < b_per_w)
            def _():
                gather_desc(c + NBUF, b).start()

            @pl.when(lax.rem(c, 64) == 63)
            def _():
                base = pl.multiple_of(wid * b_per_w + c - 63, 64)
                pltpu.sync_copy(out_v, out_hbm.at[pl.ds(base, 64)])

        def loop_body(i, carry):
            for b in range(NBUF):
                chunk(i * NBUF + b, b)
            return carry

        lax.fori_loop(0, b_per_w // NBUF, loop_body, 0)

    return k(emb, x_pad)


def _tc_mlp(s, W_in, b_in, W_h, b_h, W_out_p, b_out_p):
    BLK = 4096

    def mlp(s_ref, wi, bi, wh, bh, wo, bo, o_ref):
        h = jnp.sqrt(s_ref[...] * float(D))
        h = jnp.maximum(
            jnp.dot(h, wi[...], preferred_element_type=jnp.float32) + bi[...],
            0.0)
        h = jnp.maximum(
            jnp.dot(h, wh[...], preferred_element_type=jnp.float32) + bh[...],
            0.0)
        o = jnp.dot(h, wo[...], preferred_element_type=jnp.float32) + bo[...]
        o_ref[...] = o[:, :N_CLASSES]

    return pl.pallas_call(
        mlp,
        grid=(B // BLK,),
        in_specs=[
            pl.BlockSpec((BLK, D), lambda i: (i, 0)),
            pl.BlockSpec((D, H), lambda i: (0, 0)),
            pl.BlockSpec((1, H), lambda i: (0, 0)),
            pl.BlockSpec((H, H), lambda i: (0, 0)),
            pl.BlockSpec((1, H), lambda i: (0, 0)),
            pl.BlockSpec((H, 128), lambda i: (0, 0)),
            pl.BlockSpec((1, 128), lambda i: (0, 0)),
        ],
        out_specs=pl.BlockSpec((BLK, N_CLASSES), lambda i: (i, 0)),
        out_shape=jax.ShapeDtypeStruct((B, N_CLASSES), jnp.float32),
    )(s, W_in, b_in.reshape(1, H), W_h, b_h.reshape(1, H),
      W_out_p, b_out_p)


def kernel(x, emb, W_in, b_in, W_h, b_h, W_out, b_out):
    x32 = x.astype(jnp.int32)
    x_pad = jnp.pad(x32, ((0, 0), (0, 128 - HIST)))
    s = _sc_sumsq(emb, x_pad)
    W_out_p = jnp.pad(W_out, ((0, 0), (0, 128 - N_CLASSES)))
    b_out_p = jnp.pad(b_out, (0, 128 - N_CLASSES)).reshape(1, 128)
    return _tc_mlp(s, W_in, b_in, W_h, b_h, W_out_p, b_out_p)
